# baseline (speedup 1.0000x reference)
import jax
import jax.numpy as jnp
from jax import lax
from jax.experimental import pallas as pl

DIM = 256
N_EMBED = 8192
BM = 256
BN = 2048


def _dist_body(x_ref, e_ref, dist_ref, pmax_ref):
    j = pl.program_id(1)
    x = x_ref[...]
    e = e_ref[...]
    a = jnp.sum(x * x, axis=1, keepdims=True)
    b = jnp.sum(e * e, axis=0, keepdims=True)
    m = lax.dot_general(x, e, (((1,), (0,)), ((), ())),
                        preferred_element_type=jnp.float32)
    dist = a - 2.0 * m + b
    dist_ref[...] = dist
    pm = jnp.max(1.0 / dist, axis=1, keepdims=True)

    @pl.when(j == 0)
    def _():
        pmax_ref[...] = pm

    @pl.when(j > 0)
    def _():
        pmax_ref[...] = jnp.maximum(pmax_ref[...], pm)


def _dist_pmax(flatten, embed):
    n = flatten.shape[0]
    return pl.pallas_call(
        _dist_body,
        grid=(n // BM, N_EMBED // BN),
        in_specs=[
            pl.BlockSpec((BM, DIM), lambda i, j: (i, 0)),
            pl.BlockSpec((DIM, BN), lambda i, j: (0, j)),
        ],
        out_specs=[
            pl.BlockSpec((BM, BN), lambda i, j: (i, j)),
            pl.BlockSpec((BM, 1), lambda i, j: (i, 0)),
        ],
        out_shape=[
            jax.ShapeDtypeStruct((n, N_EMBED), jnp.float32),
            jax.ShapeDtypeStruct((n, 1), jnp.float32),
        ],
    )(flatten, embed)


def kernel(input_lr, embed):
    flatten = input_lr.reshape(-1, DIM)
    dist, _pmax = _dist_pmax(flatten, embed)

    p = 1.0 / dist
    selected = jnp.argsort(-p, axis=-1)[:, :64]
    p_sel = jnp.take_along_axis(p, selected, axis=1)
    p_sel = p_sel / jnp.sum(p_sel, axis=-1, keepdims=True)
    skey = jax.random.key(42)
    sample = jax.random.categorical(skey, jnp.log(jnp.clip(p_sel, 1e-20, None)), axis=-1)
    embed_ind = jnp.take_along_axis(selected, sample[:, None], axis=1).squeeze(-1)
    embed_ind = embed_ind.reshape(input_lr.shape[:-1])
    quantize = jnp.take(embed.T, embed_ind, axis=0)
    diff = jnp.mean((jax.lax.stop_gradient(quantize) - input_lr) ** 2)
    quantize = input_lr + jax.lax.stop_gradient(quantize - input_lr)
    return (quantize, diff, embed_ind, dist)


# speedup vs baseline: 4.7140x; 4.7140x over previous
import functools

import jax
import jax.numpy as jnp
from jax import lax
from jax.experimental import pallas as pl
from jax.experimental.pallas import tpu as pltpu
from jax.experimental.pallas import tpu_sc as plsc

DIM = 256
NE = 8192
N = 4608
BM = 256
BN = 2048
NW = 32
RPW = N // NW
CAP = 256
PRUNE_AT = 240
K = 64
CHUNK = 48


def _dist_body(x_ref, e_ref, dist_ref, pmax_ref):
    j = pl.program_id(1)
    x = x_ref[...]
    e = e_ref[...]
    a = jnp.sum(x * x, axis=1, keepdims=True)
    b = jnp.sum(e * e, axis=0, keepdims=True)
    m = lax.dot_general(x, e, (((1,), (0,)), ((), ())),
                        preferred_element_type=jnp.float32)
    dist = a - 2.0 * m + b
    dist_ref[...] = dist
    pm = jnp.max(1.0 / dist, axis=1, keepdims=True)

    @pl.when(j == 0)
    def _():
        pmax_ref[...] = pm

    @pl.when(j > 0)
    def _():
        pmax_ref[...] = jnp.maximum(pmax_ref[...], pm)


def _dist_pmax(flatten, embed):
    return pl.pallas_call(
        _dist_body,
        grid=(N // BM, NE // BN),
        in_specs=[
            pl.BlockSpec((BM, DIM), lambda i, j: (i, 0)),
            pl.BlockSpec((DIM, BN), lambda i, j: (0, j)),
        ],
        out_specs=[
            pl.BlockSpec((BM, BN), lambda i, j: (i, j)),
            pl.BlockSpec((BM, 1), lambda i, j: (i, 0)),
        ],
        out_shape=[
            jax.ShapeDtypeStruct((N, NE), jnp.float32),
            jax.ShapeDtypeStruct((N, 1), jnp.float32),
        ],
    )(flatten, embed)


def _vsort_desc(v):
    k, _ = plsc.sort_key_val(v, v, descending=True)
    return k


def _clean_desc(vs):
    n = len(vs)
    if n == 1:
        return [_vsort_desc(vs[0])]
    half = n // 2
    lo = [jnp.maximum(vs[i], vs[i + half]) for i in range(half)]
    hi = [jnp.minimum(vs[i], vs[i + half]) for i in range(half)]
    return _clean_desc(lo) + _clean_desc(hi)


def _merge_desc(a, b):
    brev = [lax.rev(x, (0,)) for x in reversed(b)]
    return _clean_desc(a + brev)


def _sort_desc(vs):
    runs = [[_vsort_desc(v)] for v in vs]
    while len(runs) > 1:
        nxt = []
        for i in range(0, len(runs), 2):
            nxt.append(_merge_desc(runs[i], runs[i + 1]) if i + 1 < len(runs) else runs[i])
        runs = nxt
    return runs[0]


def _topk_body(dist_hbm, pmax_hbm, psel_hbm, sel_hbm,
               drow, pmaxv, cand, psel_st, sel_st):
    wid = lax.axis_index("s") * 2 + lax.axis_index("c")
    row0 = wid * RPW
    iota = lax.iota(jnp.int32, 16)

    pltpu.sync_copy(pmax_hbm.at[pl.ds(row0, RPW)], pmaxv)

    def sort_cand(cnt):
        vs = []
        for vi in range(CAP // 16):
            v = cand[pl.ds(vi * 16, 16)]
            lane = iota + (vi * 16)
            v = jnp.where(lane < cnt, v, 0)
            vs.append(v)
        return _sort_desc(vs)

    def do_row(r, _):
        pltpu.sync_copy(dist_hbm.at[r + row0], drow)
        g = r // 16
        t = r % 16
        mvreg = pmaxv[pl.ds(g * 16, 16)]
        m_splat = jnp.take(mvreg, jnp.full((16,), t, jnp.int32))
        moffs = plsc.bitcast(m_splat, jnp.int32) - 8191

        def scan_step(k, carry):
            cnt, tau = carry
            d = drow[pl.ds(k * 16, 16)]
            p = 1.0 / d
            msk = p >= tau
            inc = jnp.sum(jnp.where(msk, 1, 0))
            ukey = plsc.bitcast(p, jnp.int32)
            val = jnp.maximum(ukey - moffs, 0)
            q = jnp.bitwise_or(jnp.left_shift(val, 13),
                               jnp.full((16,), 8191 - k * 16, jnp.int32) - iota)
            plsc.store_compressed(cand.at[pl.ds(cnt, 16)], q, mask=msk)
            cnt = cnt + inc

            def prune(args):
                cnt, tau = args
                top = sort_cand(cnt)
                for vi in range(K // 16):
                    cand[pl.ds(vi * 16, 16)] = top[vi]
                kth = jnp.min(top[K // 16 - 1])
                tau_i = lax.shift_right_logical(kth, 13) + moffs
                return jnp.int32(K), plsc.bitcast(tau_i, jnp.float32)

            cnt, tau = lax.cond(cnt >= PRUNE_AT, prune, lambda args: args,
                                (cnt, tau))
            return cnt, tau

        cnt, _tau = lax.fori_loop(0, NE // 16, scan_step,
                                  (jnp.int32(0), jnp.zeros((16,), jnp.float32)))

        top = sort_cand(cnt)
        for vi in range(K // 16):
            q = top[vi]
            idxv = jnp.full((16,), 8191, jnp.int32) - jnp.bitwise_and(q, 8191)
            pv = plsc.bitcast(lax.shift_right_logical(q, 13) + moffs,
                              jnp.float32)
            psel_st[pl.ds(r * K + vi * 16, 16)] = pv
            sel_st[pl.ds(r * K + vi * 16, 16)] = idxv
        return 0

    lax.fori_loop(0, RPW, do_row, 0)
    pltpu.sync_copy(psel_st, psel_hbm.at[pl.ds(row0 * K, RPW * K)])
    pltpu.sync_copy(sel_st, sel_hbm.at[pl.ds(row0 * K, RPW * K)])


def _topk_sc(dist, pmax):
    mesh = plsc.VectorSubcoreMesh(core_axis_name="c", subcore_axis_name="s")
    f = functools.partial(
        pl.kernel, mesh=mesh,
        compiler_params=pltpu.CompilerParams(needs_layout_passes=False),
        out_type=(jax.ShapeDtypeStruct((N * K,), jnp.float32),
                  jax.ShapeDtypeStruct((N * K,), jnp.int32)),
        scratch_types=[
            pltpu.VMEM((NE,), jnp.float32),
            pltpu.VMEM((RPW,), jnp.float32),
            pltpu.VMEM((CAP,), jnp.int32),
            pltpu.VMEM((RPW * K,), jnp.float32),
            pltpu.VMEM((RPW * K,), jnp.int32),
        ],
    )(_topk_body)
    psel, sel = f(dist, pmax)
    return psel.reshape(N, K), sel.reshape(N, K)


def _sample_body(psel_ref, sel_ref, g_ref, out_ref):
    psel = psel_ref[...]
    s = jnp.sum(psel, axis=1, keepdims=True)
    pn = psel / s
    logits = jnp.log(jnp.clip(pn, 1e-20, None))
    w = logits + g_ref[...]
    mx = jnp.max(w, axis=1, keepdims=True)
    iota = lax.broadcasted_iota(jnp.int32, w.shape, 1)
    samp = jnp.min(jnp.where(w == mx, iota, K), axis=1, keepdims=True)
    ind = jnp.sum(jnp.where(iota == samp, sel_ref[...], 0), axis=1,
                  keepdims=True)
    out_ref[...] = ind


def _sample(psel, sel, gum):
    return pl.pallas_call(
        _sample_body,
        out_shape=jax.ShapeDtypeStruct((N, 1), jnp.int32),
    )(psel, sel, gum)


def _gather_body(ind_hbm, embt_hbm, x_hbm, q_hbm, part_hbm,
                 idxv, rows, xv, qv, accv, sem):
    wid = lax.axis_index("s") * 2 + lax.axis_index("c")

    def chunk(c, acc):
        base = wid * RPW + c * CHUNK
        pltpu.sync_copy(ind_hbm.at[pl.ds(base, CHUNK)], idxv)
        pltpu.async_copy(embt_hbm.at[idxv], rows, sem).wait()
        pltpu.sync_copy(x_hbm.at[pl.ds(base * DIM, CHUNK * DIM)], xv)

        def vstep(k, acc):
            r = k // 16
            seg = k % 16
            qrow = rows[r, pl.ds(seg * 16, 16)]
            x = xv[pl.ds(k * 16, 16)]
            dlt = qrow - x
            qv[pl.ds(k * 16, 16)] = x + dlt
            return acc + dlt * dlt

        acc = lax.fori_loop(0, CHUNK * 16, vstep, acc)
        pltpu.sync_copy(qv, q_hbm.at[pl.ds(base * DIM, CHUNK * DIM)])
        return acc

    acc = lax.fori_loop(0, RPW // CHUNK, chunk, jnp.zeros((16,), jnp.float32))
    accv[...] = acc
    pltpu.sync_copy(accv, part_hbm.at[wid])


def _gather_sc(ind, embt, xflat):
    mesh = plsc.VectorSubcoreMesh(core_axis_name="c", subcore_axis_name="s")
    f = functools.partial(
        pl.kernel, mesh=mesh,
        compiler_params=pltpu.CompilerParams(needs_layout_passes=False),
        out_type=(jax.ShapeDtypeStruct((N * DIM,), jnp.float32),
                  jax.ShapeDtypeStruct((NW, 16), jnp.float32)),
        scratch_types=[
            pltpu.VMEM((CHUNK,), jnp.int32),
            pltpu.VMEM((CHUNK, DIM), jnp.float32),
            pltpu.VMEM((CHUNK * DIM,), jnp.float32),
            pltpu.VMEM((CHUNK * DIM,), jnp.float32),
            pltpu.VMEM((16,), jnp.float32),
            pltpu.SemaphoreType.DMA,
        ],
    )(_gather_body)
    return f(ind, embt, xflat)


def kernel(input_lr, embed):
    flatten = input_lr.reshape(-1, DIM)
    dist, pmax = _dist_pmax(flatten, embed)
    psel, sel = _topk_sc(dist, pmax.reshape(-1))
    gum = jax.random.gumbel(jax.random.key(42), (N, K), jnp.float32)
    ind = _sample(psel, sel, gum)
    embt = embed.T
    qflat, partials = _gather_sc(ind.reshape(-1), embt, flatten.reshape(-1))
    quantize = qflat.reshape(input_lr.shape)
    diff = jnp.sum(partials) / jnp.float32(N * DIM)
    embed_ind = ind.reshape(input_lr.shape[:-1])
    return (quantize, diff, embed_ind, dist)


# speedup vs baseline: 8.0810x; 1.7142x over previous
import functools

import jax
import jax.numpy as jnp
from jax import lax
from jax.experimental import pallas as pl
from jax.experimental.pallas import tpu as pltpu
from jax.experimental.pallas import tpu_sc as plsc

DIM = 256
NE = 8192
N = 4608
BM = 256
BN = 2048
NW = 32
RPW = N // NW
CAP = 512
K = 64
CHUNK = 48


def _dist_body(x_ref, e_ref, dist_ref, pmax_ref, tau0_ref):
    j = pl.program_id(1)
    x = x_ref[...]
    e = e_ref[...]
    a = jnp.sum(x * x, axis=1, keepdims=True)
    b = jnp.sum(e * e, axis=0, keepdims=True)
    m = lax.dot_general(x, e, (((1,), (0,)), ((), ())),
                        preferred_element_type=jnp.float32)
    dist = a - 2.0 * m + b
    dist_ref[...] = dist
    p = 1.0 / dist
    pm = jnp.max(p, axis=1, keepdims=True)
    cm = jnp.max(p.reshape(BM, BN // 128, 128), axis=2)
    tb = jnp.min(cm, axis=1, keepdims=True)

    @pl.when(j == 0)
    def _():
        pmax_ref[...] = pm
        tau0_ref[...] = tb

    @pl.when(j > 0)
    def _():
        pmax_ref[...] = jnp.maximum(pmax_ref[...], pm)
        tau0_ref[...] = jnp.minimum(tau0_ref[...], tb)


def _dist_pmax(flatten, embed):
    return pl.pallas_call(
        _dist_body,
        grid=(N // BM, NE // BN),
        in_specs=[
            pl.BlockSpec((BM, DIM), lambda i, j: (i, 0)),
            pl.BlockSpec((DIM, BN), lambda i, j: (0, j)),
        ],
        out_specs=[
            pl.BlockSpec((BM, BN), lambda i, j: (i, j)),
            pl.BlockSpec((BM, 1), lambda i, j: (i, 0)),
            pl.BlockSpec((BM, 1), lambda i, j: (i, 0)),
        ],
        out_shape=[
            jax.ShapeDtypeStruct((N, NE), jnp.float32),
            jax.ShapeDtypeStruct((N, 1), jnp.float32),
            jax.ShapeDtypeStruct((N, 1), jnp.float32),
        ],
    )(flatten, embed)


def _vsort_desc(v):
    k, _ = plsc.sort_key_val(v, v, descending=True)
    return k


def _clean_desc(vs):
    n = len(vs)
    if n == 1:
        return [_vsort_desc(vs[0])]
    half = n // 2
    lo = [jnp.maximum(vs[i], vs[i + half]) for i in range(half)]
    hi = [jnp.minimum(vs[i], vs[i + half]) for i in range(half)]
    return _clean_desc(lo) + _clean_desc(hi)


def _merge_desc(a, b):
    brev = [lax.rev(x, (0,)) for x in reversed(b)]
    return _clean_desc(a + brev)


def _sort_desc(vs):
    runs = [[_vsort_desc(v)] for v in vs]
    while len(runs) > 1:
        nxt = []
        for i in range(0, len(runs), 2):
            nxt.append(_merge_desc(runs[i], runs[i + 1]) if i + 1 < len(runs) else runs[i])
        runs = nxt
    return runs[0]


def _sort_top4(vs):
    runs = [[_vsort_desc(v)] for v in vs]
    while len(runs) > 1:
        nxt = []
        for i in range(0, len(runs), 2):
            if i + 1 == len(runs):
                nxt.append(runs[i])
                continue
            a, b = runs[i], runs[i + 1]
            if len(a) < 4 or len(b) < 4:
                nxt.append(_merge_desc(a, b))
            else:
                brev = [lax.rev(x, (0,)) for x in reversed(b)]
                lo = [jnp.maximum(a[j], brev[j]) for j in range(4)]
                nxt.append(_clean_desc(lo))
        runs = nxt
    return runs[0][:4]


UNROLL = 8


def _topk_body(dist_hbm, pmax_hbm, tau0_hbm, psel_hbm, sel_hbm,
               dbuf0, dbuf1, pmaxv, tauv, cand, psel_st, sel_st,
               sem_a, sem_b):
    wid = lax.axis_index("s") * 2 + lax.axis_index("c")
    row0 = wid * RPW
    iota = lax.iota(jnp.int32, 16)

    pltpu.sync_copy(pmax_hbm.at[pl.ds(row0, RPW)], pmaxv)
    pltpu.sync_copy(tau0_hbm.at[pl.ds(row0, RPW)], tauv)

    def start(r, buf, sem):
        pltpu.async_copy(dist_hbm.at[r + row0], buf, sem)

    def wait(r, buf, sem):
        pltpu.make_async_copy(dist_hbm.at[r + row0], buf, sem).wait()

    def sort_cand(cnt):
        vs = []
        for vi in range(CAP // 16):
            v = cand[pl.ds(vi * 16, 16)]
            lane = iota + (vi * 16)
            v = jnp.where(lane < cnt, v, 0)
            vs.append(v)
        return _sort_top4(vs)

    def process(r, buf):
        g = r // 16
        t = r % 16
        tsplat = jnp.full((16,), t, jnp.int32)
        m_splat = jnp.take(pmaxv[pl.ds(g * 16, 16)], tsplat)
        moffs = plsc.bitcast(m_splat, jnp.int32) - 8191
        tau0 = jnp.take(tauv[pl.ds(g * 16, 16)], tsplat)

        def group(cc, carry):
            cnt, tau = carry
            base = cc * (UNROLL * 16)
            for u in range(UNROLL):
                d = buf[pl.ds(base + u * 16, 16)]
                p = 1.0 / d
                msk = p >= tau
                ukey = plsc.bitcast(p, jnp.int32)
                val = jnp.maximum(ukey - moffs, 0)
                rev = jnp.full((16,), 8191 - u * 16, jnp.int32) - base - iota
                q = jnp.bitwise_or(jnp.left_shift(val, 13), rev)
                plsc.store_compressed(cand.at[pl.ds(cnt, 16)], q, mask=msk)
                cnt = cnt + plsc.all_reduce_population_count(msk)[0]

            def prune(args):
                cnt, tau = args
                top = sort_cand(cnt)
                for vi in range(K // 16):
                    cand[pl.ds(vi * 16, 16)] = top[vi]
                kth = jnp.min(top[K // 16 - 1])
                tau_i = lax.shift_right_logical(kth, 13) + moffs
                return jnp.int32(K), plsc.bitcast(tau_i, jnp.float32)

            cnt, tau = lax.cond(cnt >= CAP - UNROLL * 16, prune,
                                lambda args: args, (cnt, tau))
            return cnt, tau

        cnt, _tau = lax.fori_loop(0, NE // (UNROLL * 16), group,
                                  (jnp.int32(0), tau0))

        top = sort_cand(cnt)
        for vi in range(K // 16):
            q = top[vi]
            idxv = jnp.full((16,), 8191, jnp.int32) - jnp.bitwise_and(q, 8191)
            pv = plsc.bitcast(lax.shift_right_logical(q, 13) + moffs,
                              jnp.float32)
            psel_st[pl.ds(r * K + vi * 16, 16)] = pv
            sel_st[pl.ds(r * K + vi * 16, 16)] = idxv

    start(0, dbuf0, sem_a)

    def outer(gg, _):
        r0 = 2 * gg
        r1 = r0 + 1
        wait(r0, dbuf0, sem_a)
        start(r1, dbuf1, sem_b)
        process(r0, dbuf0)
        wait(r1, dbuf1, sem_b)

        @pl.when(r1 + 1 < RPW)
        def _():
            start(r1 + 1, dbuf0, sem_a)

        process(r1, dbuf1)
        return 0

    lax.fori_loop(0, RPW // 2, outer, 0)
    pltpu.sync_copy(psel_st, psel_hbm.at[pl.ds(row0 * K, RPW * K)])
    pltpu.sync_copy(sel_st, sel_hbm.at[pl.ds(row0 * K, RPW * K)])


def _topk_sc(dist, pmax, tau0):
    mesh = plsc.VectorSubcoreMesh(core_axis_name="c", subcore_axis_name="s")
    f = functools.partial(
        pl.kernel, mesh=mesh,
        compiler_params=pltpu.CompilerParams(needs_layout_passes=False),
        out_type=(jax.ShapeDtypeStruct((N * K,), jnp.float32),
                  jax.ShapeDtypeStruct((N * K,), jnp.int32)),
        scratch_types=[
            pltpu.VMEM((NE,), jnp.float32),
            pltpu.VMEM((NE,), jnp.float32),
            pltpu.VMEM((RPW,), jnp.float32),
            pltpu.VMEM((RPW,), jnp.float32),
            pltpu.VMEM((CAP,), jnp.int32),
            pltpu.VMEM((RPW * K,), jnp.float32),
            pltpu.VMEM((RPW * K,), jnp.int32),
            pltpu.SemaphoreType.DMA,
            pltpu.SemaphoreType.DMA,
        ],
    )(_topk_body)
    psel, sel = f(dist, pmax, tau0)
    return psel.reshape(N, K), sel.reshape(N, K)


def _sample_body(psel_ref, sel_ref, g_ref, out_ref):
    psel = psel_ref[...]
    s = jnp.sum(psel, axis=1, keepdims=True)
    pn = psel / s
    logits = jnp.log(jnp.clip(pn, 1e-20, None))
    w = logits + g_ref[...]
    mx = jnp.max(w, axis=1, keepdims=True)
    iota = lax.broadcasted_iota(jnp.int32, w.shape, 1)
    samp = jnp.min(jnp.where(w == mx, iota, K), axis=1, keepdims=True)
    ind = jnp.sum(jnp.where(iota == samp, sel_ref[...], 0), axis=1,
                  keepdims=True)
    out_ref[...] = ind


def _sample(psel, sel, gum):
    return pl.pallas_call(
        _sample_body,
        out_shape=jax.ShapeDtypeStruct((N, 1), jnp.int32),
    )(psel, sel, gum)


def _gather_body(ind_hbm, embt_hbm, x_hbm, q_hbm, part_hbm,
                 idxv, rows, xv, qv, accv, sem):
    wid = lax.axis_index("s") * 2 + lax.axis_index("c")

    def chunk(c, acc):
        base = wid * RPW + c * CHUNK
        pltpu.sync_copy(ind_hbm.at[pl.ds(base, CHUNK)], idxv)
        pltpu.async_copy(embt_hbm.at[idxv], rows, sem).wait()
        pltpu.sync_copy(x_hbm.at[pl.ds(base * DIM, CHUNK * DIM)], xv)

        def vstep(k, acc):
            r = k // 16
            seg = k % 16
            qrow = rows[r, pl.ds(seg * 16, 16)]
            x = xv[pl.ds(k * 16, 16)]
            dlt = qrow - x
            qv[pl.ds(k * 16, 16)] = x + dlt
            return acc + dlt * dlt

        acc = lax.fori_loop(0, CHUNK * 16, vstep, acc)
        pltpu.sync_copy(qv, q_hbm.at[pl.ds(base * DIM, CHUNK * DIM)])
        return acc

    acc = lax.fori_loop(0, RPW // CHUNK, chunk, jnp.zeros((16,), jnp.float32))
    accv[...] = acc
    pltpu.sync_copy(accv, part_hbm.at[wid])


def _gather_sc(ind, embt, xflat):
    mesh = plsc.VectorSubcoreMesh(core_axis_name="c", subcore_axis_name="s")
    f = functools.partial(
        pl.kernel, mesh=mesh,
        compiler_params=pltpu.CompilerParams(needs_layout_passes=False),
        out_type=(jax.ShapeDtypeStruct((N * DIM,), jnp.float32),
                  jax.ShapeDtypeStruct((NW, 16), jnp.float32)),
        scratch_types=[
            pltpu.VMEM((CHUNK,), jnp.int32),
            pltpu.VMEM((CHUNK, DIM), jnp.float32),
            pltpu.VMEM((CHUNK * DIM,), jnp.float32),
            pltpu.VMEM((CHUNK * DIM,), jnp.float32),
            pltpu.VMEM((16,), jnp.float32),
            pltpu.SemaphoreType.DMA,
        ],
    )(_gather_body)
    return f(ind, embt, xflat)


def kernel(input_lr, embed):
    flatten = input_lr.reshape(-1, DIM)
    dist, pmax, tau0 = _dist_pmax(flatten, embed)
    psel, sel = _topk_sc(dist, pmax.reshape(-1), tau0.reshape(-1))
    gum = jax.random.gumbel(jax.random.key(42), (N, K), jnp.float32)
    ind = _sample(psel, sel, gum)
    embt = embed.T
    qflat, partials = _gather_sc(ind.reshape(-1), embt, flatten.reshape(-1))
    quantize = qflat.reshape(input_lr.shape)
    diff = jnp.sum(partials) / jnp.float32(N * DIM)
    embed_ind = ind.reshape(input_lr.shape[:-1])
    return (quantize, diff, embed_ind, dist)


# speedup vs baseline: 12.1366x; 1.5019x over previous
import functools

import jax
import jax.numpy as jnp
from jax import lax
from jax.experimental import pallas as pl
from jax.experimental.pallas import tpu as pltpu
from jax.experimental.pallas import tpu_sc as plsc

DIM = 256
NE = 8192
N = 4608
BM = 256
BN = 2048
NW = 32
RPW = N // NW
CAP = 512
K = 64
CHUNK = 48


def _dist_body(x_ref, e_ref, dist_ref, pmax_ref, tau0_ref):
    j = pl.program_id(1)
    x = x_ref[...]
    e = e_ref[...]
    a = jnp.sum(x * x, axis=1, keepdims=True)
    b = jnp.sum(e * e, axis=0, keepdims=True)
    m = lax.dot_general(x, e, (((1,), (0,)), ((), ())),
                        preferred_element_type=jnp.float32)
    dist = a - 2.0 * m + b
    dist_ref[...] = dist
    pm = jnp.max(1.0 / dist, axis=1, keepdims=True)
    cm = jnp.min(dist.reshape(BM, BN // 128, 128), axis=2)
    tb = jnp.max(cm, axis=1, keepdims=True)

    @pl.when(j == 0)
    def _():
        pmax_ref[...] = pm
        tau0_ref[...] = tb

    @pl.when(j > 0)
    def _():
        pmax_ref[...] = jnp.maximum(pmax_ref[...], pm)
        tau0_ref[...] = jnp.maximum(tau0_ref[...], tb)


def _dist_pmax(flatten, embed):
    return pl.pallas_call(
        _dist_body,
        grid=(N // BM, NE // BN),
        in_specs=[
            pl.BlockSpec((BM, DIM), lambda i, j: (i, 0)),
            pl.BlockSpec((DIM, BN), lambda i, j: (0, j)),
        ],
        out_specs=[
            pl.BlockSpec((BM, BN), lambda i, j: (i, j)),
            pl.BlockSpec((BM, 1), lambda i, j: (i, 0)),
            pl.BlockSpec((BM, 1), lambda i, j: (i, 0)),
        ],
        out_shape=[
            jax.ShapeDtypeStruct((N, NE), jnp.float32),
            jax.ShapeDtypeStruct((N, 1), jnp.float32),
            jax.ShapeDtypeStruct((N, 1), jnp.float32),
        ],
    )(flatten, embed)


def _vsort_desc(v):
    k, _ = plsc.sort_key_val(v, v, descending=True)
    return k


def _clean_desc(vs):
    n = len(vs)
    if n == 1:
        return [_vsort_desc(vs[0])]
    half = n // 2
    lo = [jnp.maximum(vs[i], vs[i + half]) for i in range(half)]
    hi = [jnp.minimum(vs[i], vs[i + half]) for i in range(half)]
    return _clean_desc(lo) + _clean_desc(hi)


def _merge_desc(a, b):
    brev = [lax.rev(x, (0,)) for x in reversed(b)]
    return _clean_desc(a + brev)


def _sort_desc(vs):
    runs = [[_vsort_desc(v)] for v in vs]
    while len(runs) > 1:
        nxt = []
        for i in range(0, len(runs), 2):
            nxt.append(_merge_desc(runs[i], runs[i + 1]) if i + 1 < len(runs) else runs[i])
        runs = nxt
    return runs[0]


def _sort_top4(vs):
    runs = [[_vsort_desc(v)] for v in vs]
    while len(runs) > 1:
        nxt = []
        for i in range(0, len(runs), 2):
            if i + 1 == len(runs):
                nxt.append(runs[i])
                continue
            a, b = runs[i], runs[i + 1]
            if len(a) < 4 or len(b) < 4:
                nxt.append(_merge_desc(a, b))
            else:
                brev = [lax.rev(x, (0,)) for x in reversed(b)]
                lo = [jnp.maximum(a[j], brev[j]) for j in range(4)]
                nxt.append(_clean_desc(lo))
        runs = nxt
    return runs[0][:4]


UNROLL = 8
CAP_C = 256


def _topk_body(dist_hbm, pmax_hbm, tau0_hbm, psel_hbm, sel_hbm,
               dbuf0, dbuf1, pmaxv, tauv, cand_a, cand_b, psel_st, sel_st,
               sem_a, sem_b):
    wid = lax.axis_index("s") * 2 + lax.axis_index("c")
    row0 = wid * RPW
    iota = lax.iota(jnp.int32, 16)
    zero16 = jnp.zeros((16,), jnp.int32)

    for vi in range(CAP_C // 16):
        cand_a[pl.ds(vi * 16, 16)] = zero16
        cand_b[pl.ds(vi * 16, 16)] = zero16

    pltpu.sync_copy(pmax_hbm.at[pl.ds(row0, RPW)], pmaxv)
    pltpu.sync_copy(tau0_hbm.at[pl.ds(row0, RPW)], tauv)

    def start(r, buf, sem):
        pltpu.async_copy(dist_hbm.at[r + row0], buf, sem)

    def wait(r, buf, sem):
        pltpu.make_async_copy(dist_hbm.at[r + row0], buf, sem).wait()

    def process(r, buf):
        g = r // 16
        t = r % 16
        tsplat = jnp.full((16,), t, jnp.int32)
        m_splat = jnp.take(pmaxv[pl.ds(g * 16, 16)], tsplat)
        moffs = plsc.bitcast(m_splat, jnp.int32) - 8191
        taud_raw = jnp.take(tauv[pl.ds(g * 16, 16)], tsplat)
        tau0 = plsc.bitcast(plsc.bitcast(taud_raw, jnp.int32) + 8,
                            jnp.float32)

        def build_packed(cand, cnt):
            vs = []
            for vi in range(CAP_C // 16):
                idxv = cand[pl.ds(vi * 16, 16)]
                d = plsc.load_gather(buf, [idxv])
                p = 1.0 / d
                ukey = plsc.bitcast(p, jnp.int32)
                val = jnp.maximum(ukey - moffs, 0)
                q = jnp.bitwise_or(jnp.left_shift(val, 13),
                                   jnp.full((16,), 8191, jnp.int32) - idxv)
                lane = iota + (vi * 16)
                vs.append(jnp.where(lane < cnt, q, 0))
            return vs

        def make_prune(cand):
            def prune(args):
                cnt, tau = args
                top = _sort_top4(build_packed(cand, cnt))
                for vi in range(K // 16):
                    cand[pl.ds(vi * 16, 16)] = (
                        jnp.full((16,), 8191, jnp.int32)
                        - jnp.bitwise_and(top[vi], 8191))
                p64 = plsc.bitcast(
                    lax.shift_right_logical(jnp.min(top[K // 16 - 1]), 13)
                    + moffs, jnp.float32)
                tau_new = plsc.bitcast(
                    plsc.bitcast(1.0 / p64, jnp.int32) + 8, jnp.float32)
                return jnp.int32(K), tau_new
            return prune

        prune_a = make_prune(cand_a)
        prune_b = make_prune(cand_b)

        def group(cc, carry):
            cnt_a, tau_a, cnt_b, tau_b = carry
            base = cc * (UNROLL * 16)
            for u in range(UNROLL // 2):
                d = buf[pl.ds(base + u * 16, 16)]
                msk = d <= tau_a
                idxv = jnp.full((16,), u * 16, jnp.int32) + base + iota
                plsc.store_compressed(cand_a.at[pl.ds(cnt_a, 16)], idxv,
                                      mask=msk)
                cnt_a = cnt_a + plsc.all_reduce_population_count(msk)[0]
            for u in range(UNROLL // 2, UNROLL):
                d = buf[pl.ds(base + u * 16, 16)]
                msk = d <= tau_b
                idxv = jnp.full((16,), u * 16, jnp.int32) + base + iota
                plsc.store_compressed(cand_b.at[pl.ds(cnt_b, 16)], idxv,
                                      mask=msk)
                cnt_b = cnt_b + plsc.all_reduce_population_count(msk)[0]

            cnt_a, tau_a = lax.cond(cnt_a >= CAP_C - UNROLL * 8, prune_a,
                                    lambda args: args, (cnt_a, tau_a))
            cnt_b, tau_b = lax.cond(cnt_b >= CAP_C - UNROLL * 8, prune_b,
                                    lambda args: args, (cnt_b, tau_b))
            return cnt_a, tau_a, cnt_b, tau_b

        cnt_a, _ta, cnt_b, _tb = lax.fori_loop(
            0, NE // (UNROLL * 16), group,
            (jnp.int32(0), tau0, jnp.int32(0), tau0))

        top = _sort_top4(build_packed(cand_a, cnt_a)
                         + build_packed(cand_b, cnt_b))
        for vi in range(K // 16):
            q = top[vi]
            idxv = jnp.full((16,), 8191, jnp.int32) - jnp.bitwise_and(q, 8191)
            pv = plsc.bitcast(lax.shift_right_logical(q, 13) + moffs,
                              jnp.float32)
            psel_st[pl.ds(r * K + vi * 16, 16)] = pv
            sel_st[pl.ds(r * K + vi * 16, 16)] = idxv

    start(0, dbuf0, sem_a)

    def outer(gg, _):
        r0 = 2 * gg
        r1 = r0 + 1
        wait(r0, dbuf0, sem_a)
        start(r1, dbuf1, sem_b)
        process(r0, dbuf0)
        wait(r1, dbuf1, sem_b)

        @pl.when(r1 + 1 < RPW)
        def _():
            start(r1 + 1, dbuf0, sem_a)

        process(r1, dbuf1)
        return 0

    lax.fori_loop(0, RPW // 2, outer, 0)
    pltpu.sync_copy(psel_st, psel_hbm.at[pl.ds(row0 * K, RPW * K)])
    pltpu.sync_copy(sel_st, sel_hbm.at[pl.ds(row0 * K, RPW * K)])


def _topk_sc(dist, pmax, tau0):
    mesh = plsc.VectorSubcoreMesh(core_axis_name="c", subcore_axis_name="s")
    f = functools.partial(
        pl.kernel, mesh=mesh,
        compiler_params=pltpu.CompilerParams(needs_layout_passes=False),
        out_type=(jax.ShapeDtypeStruct((N * K,), jnp.float32),
                  jax.ShapeDtypeStruct((N * K,), jnp.int32)),
        scratch_types=[
            pltpu.VMEM((NE,), jnp.float32),
            pltpu.VMEM((NE,), jnp.float32),
            pltpu.VMEM((RPW,), jnp.float32),
            pltpu.VMEM((RPW,), jnp.float32),
            pltpu.VMEM((CAP_C,), jnp.int32),
            pltpu.VMEM((CAP_C,), jnp.int32),
            pltpu.VMEM((RPW * K,), jnp.float32),
            pltpu.VMEM((RPW * K,), jnp.int32),
            pltpu.SemaphoreType.DMA,
            pltpu.SemaphoreType.DMA,
        ],
    )(_topk_body)
    psel, sel = f(dist, pmax, tau0)
    return psel.reshape(N, K), sel.reshape(N, K)


def _sample_body(psel_ref, sel_ref, g_ref, out_ref):
    psel = psel_ref[...]
    s = jnp.sum(psel, axis=1, keepdims=True)
    pn = psel / s
    logits = jnp.log(jnp.clip(pn, 1e-20, None))
    w = logits + g_ref[...]
    mx = jnp.max(w, axis=1, keepdims=True)
    iota = lax.broadcasted_iota(jnp.int32, w.shape, 1)
    samp = jnp.min(jnp.where(w == mx, iota, K), axis=1, keepdims=True)
    ind = jnp.sum(jnp.where(iota == samp, sel_ref[...], 0), axis=1,
                  keepdims=True)
    out_ref[...] = ind


def _sample(psel, sel, gum):
    return pl.pallas_call(
        _sample_body,
        out_shape=jax.ShapeDtypeStruct((N, 1), jnp.int32),
    )(psel, sel, gum)


def _gather_body(ind_hbm, embt_hbm, x_hbm, q_hbm, part_hbm,
                 idxv, rows, xv, qv, accv, sem):
    wid = lax.axis_index("s") * 2 + lax.axis_index("c")

    def chunk(c, acc):
        base = wid * RPW + c * CHUNK
        pltpu.sync_copy(ind_hbm.at[pl.ds(base, CHUNK)], idxv)
        pltpu.async_copy(embt_hbm.at[idxv], rows, sem).wait()
        pltpu.sync_copy(x_hbm.at[pl.ds(base * DIM, CHUNK * DIM)], xv)

        def vstep(k, acc):
            r = k // 16
            seg = k % 16
            qrow = rows[r, pl.ds(seg * 16, 16)]
            x = xv[pl.ds(k * 16, 16)]
            dlt = qrow - x
            qv[pl.ds(k * 16, 16)] = x + dlt
            return acc + dlt * dlt

        acc = lax.fori_loop(0, CHUNK * 16, vstep, acc)
        pltpu.sync_copy(qv, q_hbm.at[pl.ds(base * DIM, CHUNK * DIM)])
        return acc

    acc = lax.fori_loop(0, RPW // CHUNK, chunk, jnp.zeros((16,), jnp.float32))
    accv[...] = acc
    pltpu.sync_copy(accv, part_hbm.at[wid])


def _gather_sc(ind, embt, xflat):
    mesh = plsc.VectorSubcoreMesh(core_axis_name="c", subcore_axis_name="s")
    f = functools.partial(
        pl.kernel, mesh=mesh,
        compiler_params=pltpu.CompilerParams(needs_layout_passes=False),
        out_type=(jax.ShapeDtypeStruct((N * DIM,), jnp.float32),
                  jax.ShapeDtypeStruct((NW, 16), jnp.float32)),
        scratch_types=[
            pltpu.VMEM((CHUNK,), jnp.int32),
            pltpu.VMEM((CHUNK, DIM), jnp.float32),
            pltpu.VMEM((CHUNK * DIM,), jnp.float32),
            pltpu.VMEM((CHUNK * DIM,), jnp.float32),
            pltpu.VMEM((16,), jnp.float32),
            pltpu.SemaphoreType.DMA,
        ],
    )(_gather_body)
    return f(ind, embt, xflat)


def kernel(input_lr, embed):
    flatten = input_lr.reshape(-1, DIM)
    dist, pmax, tau0 = _dist_pmax(flatten, embed)
    psel, sel = _topk_sc(dist, pmax.reshape(-1), tau0.reshape(-1))
    gum = jax.random.gumbel(jax.random.key(42), (N, K), jnp.float32)
    ind = _sample(psel, sel, gum)
    embt = embed.T
    qflat, partials = _gather_sc(ind.reshape(-1), embt, flatten.reshape(-1))
    quantize = qflat.reshape(input_lr.shape)
    diff = jnp.sum(partials) / jnp.float32(N * DIM)
    embed_ind = ind.reshape(input_lr.shape[:-1])
    return (quantize, diff, embed_ind, dist)


# speedup vs baseline: 25.6102x; 2.1102x over previous
import functools

import jax
import jax.numpy as jnp
from jax import lax
from jax.experimental import pallas as pl
from jax.experimental.pallas import tpu as pltpu
from jax.experimental.pallas import tpu_sc as plsc

DIM = 256
NE = 8192
N = 4608
BM = 256
BN = 8192
NW = 32
RPW = N // NW
CAP = 512
K = 64
CHUNK = 48


def _dist_body(x_ref, e_ref, dist_ref, pmax_ref, tau0_ref):
    x = x_ref[...]
    e = e_ref[...]
    a = jnp.sum(x * x, axis=1, keepdims=True)
    b = jnp.sum(e * e, axis=0, keepdims=True)
    m = lax.dot_general(x, e, (((1,), (0,)), ((), ())),
                        preferred_element_type=jnp.float32)
    dist = a - 2.0 * m + b
    dist_ref[...] = dist
    pmax_ref[...] = jnp.max(1.0 / dist, axis=1, keepdims=True)
    cm = jnp.min(dist.reshape(BM, BN // 128, 128), axis=2)
    tau0_ref[...] = jnp.max(cm, axis=1, keepdims=True)


def _dist_pmax(flatten, embed):
    return pl.pallas_call(
        _dist_body,
        grid=(N // BM,),
        in_specs=[
            pl.BlockSpec((BM, DIM), lambda i: (i, 0)),
            pl.BlockSpec((DIM, BN), lambda i: (0, 0)),
        ],
        out_specs=[
            pl.BlockSpec((BM, BN), lambda i: (i, 0)),
            pl.BlockSpec((BM, 1), lambda i: (i, 0)),
            pl.BlockSpec((BM, 1), lambda i: (i, 0)),
        ],
        out_shape=[
            jax.ShapeDtypeStruct((N, NE), jnp.float32),
            jax.ShapeDtypeStruct((N, 1), jnp.float32),
            jax.ShapeDtypeStruct((N, 1), jnp.float32),
        ],
    )(flatten, embed)


def _vsort_desc(v):
    k, _ = plsc.sort_key_val(v, v, descending=True)
    return k


def _clean_desc(vs):
    n = len(vs)
    if n == 1:
        return [_vsort_desc(vs[0])]
    half = n // 2
    lo = [jnp.maximum(vs[i], vs[i + half]) for i in range(half)]
    hi = [jnp.minimum(vs[i], vs[i + half]) for i in range(half)]
    return _clean_desc(lo) + _clean_desc(hi)


def _merge_desc(a, b):
    brev = [lax.rev(x, (0,)) for x in reversed(b)]
    return _clean_desc(a + brev)


def _sort_desc(vs):
    runs = [[_vsort_desc(v)] for v in vs]
    while len(runs) > 1:
        nxt = []
        for i in range(0, len(runs), 2):
            nxt.append(_merge_desc(runs[i], runs[i + 1]) if i + 1 < len(runs) else runs[i])
        runs = nxt
    return runs[0]


def _sort_top4(vs):
    runs = [[_vsort_desc(v)] for v in vs]
    while len(runs) > 1:
        nxt = []
        for i in range(0, len(runs), 2):
            if i + 1 == len(runs):
                nxt.append(runs[i])
                continue
            a, b = runs[i], runs[i + 1]
            if len(a) < 4 or len(b) < 4:
                nxt.append(_merge_desc(a, b))
            else:
                brev = [lax.rev(x, (0,)) for x in reversed(b)]
                lo = [jnp.maximum(a[j], brev[j]) for j in range(4)]
                nxt.append(_clean_desc(lo))
        runs = nxt
    return runs[0][:4]


UNROLL = 8
CAP_C = 256


def _topk_body(dist_hbm, pmax_hbm, tau0_hbm, psel_hbm, sel_hbm,
               dbuf0, dbuf1, pmaxv, tauv, cand_a, cand_b, psel_st, sel_st,
               sem_a, sem_b):
    wid = lax.axis_index("s") * 2 + lax.axis_index("c")
    row0 = wid * RPW
    iota = lax.iota(jnp.int32, 16)
    zero16 = jnp.zeros((16,), jnp.int32)

    for vi in range(CAP_C // 16):
        cand_a[pl.ds(vi * 16, 16)] = zero16
        cand_b[pl.ds(vi * 16, 16)] = zero16

    pltpu.sync_copy(pmax_hbm.at[pl.ds(row0, RPW)], pmaxv)
    pltpu.sync_copy(tau0_hbm.at[pl.ds(row0, RPW)], tauv)

    def start(r, buf, sem):
        pltpu.async_copy(dist_hbm.at[r + row0], buf, sem)

    def wait(r, buf, sem):
        pltpu.make_async_copy(dist_hbm.at[r + row0], buf, sem).wait()

    def process(r, buf):
        g = r // 16
        t = r % 16
        tsplat = jnp.full((16,), t, jnp.int32)
        m_splat = jnp.take(pmaxv[pl.ds(g * 16, 16)], tsplat)
        moffs = plsc.bitcast(m_splat, jnp.int32) - 8191
        taud_raw = jnp.take(tauv[pl.ds(g * 16, 16)], tsplat)
        tau0 = plsc.bitcast(plsc.bitcast(taud_raw, jnp.int32) + 8,
                            jnp.float32)

        def build_packed(cand, cnt):
            vs = []
            for vi in range(CAP_C // 16):
                idxv = cand[pl.ds(vi * 16, 16)]
                d = plsc.load_gather(buf, [idxv])
                p = 1.0 / d
                ukey = plsc.bitcast(p, jnp.int32)
                val = jnp.maximum(ukey - moffs, 0)
                q = jnp.bitwise_or(jnp.left_shift(val, 13),
                                   jnp.full((16,), 8191, jnp.int32) - idxv)
                lane = iota + (vi * 16)
                vs.append(jnp.where(lane < cnt, q, 0))
            return vs

        def make_prune(cand):
            def prune(args):
                cnt, tau = args
                top = _sort_top4(build_packed(cand, cnt))
                for vi in range(K // 16):
                    cand[pl.ds(vi * 16, 16)] = (
                        jnp.full((16,), 8191, jnp.int32)
                        - jnp.bitwise_and(top[vi], 8191))
                p64 = plsc.bitcast(
                    lax.shift_right_logical(jnp.min(top[K // 16 - 1]), 13)
                    + moffs, jnp.float32)
                tau_new = plsc.bitcast(
                    plsc.bitcast(1.0 / p64, jnp.int32) + 8, jnp.float32)
                return jnp.int32(K), tau_new
            return prune

        prune_a = make_prune(cand_a)
        prune_b = make_prune(cand_b)

        def group(cc, carry):
            cnt_a, tau_a, cnt_b, tau_b = carry
            base = cc * (UNROLL * 16)
            msks, incs = [], []
            for u in range(UNROLL):
                d = buf[pl.ds(base + u * 16, 16)]
                m = d <= (tau_a if u < UNROLL // 2 else tau_b)
                msks.append(m)
                incs.append(plsc.all_reduce_population_count(m)[0])
            for u in range(UNROLL // 2):
                idxv = jnp.full((16,), u * 16, jnp.int32) + base + iota
                plsc.store_compressed(cand_a.at[pl.ds(cnt_a, 16)], idxv,
                                      mask=msks[u])
                cnt_a = cnt_a + incs[u]
            for u in range(UNROLL // 2, UNROLL):
                idxv = jnp.full((16,), u * 16, jnp.int32) + base + iota
                plsc.store_compressed(cand_b.at[pl.ds(cnt_b, 16)], idxv,
                                      mask=msks[u])
                cnt_b = cnt_b + incs[u]

            cnt_a, tau_a = lax.cond(cnt_a >= CAP_C - UNROLL * 8, prune_a,
                                    lambda args: args, (cnt_a, tau_a))
            cnt_b, tau_b = lax.cond(cnt_b >= CAP_C - UNROLL * 8, prune_b,
                                    lambda args: args, (cnt_b, tau_b))
            return cnt_a, tau_a, cnt_b, tau_b

        cnt_a, _ta, cnt_b, _tb = lax.fori_loop(
            0, NE // (UNROLL * 16), group,
            (jnp.int32(0), tau0, jnp.int32(0), tau0))

        top = _sort_top4(build_packed(cand_a, cnt_a)
                         + build_packed(cand_b, cnt_b))
        for vi in range(K // 16):
            q = top[vi]
            idxv = jnp.full((16,), 8191, jnp.int32) - jnp.bitwise_and(q, 8191)
            pv = plsc.bitcast(lax.shift_right_logical(q, 13) + moffs,
                              jnp.float32)
            psel_st[pl.ds(r * K + vi * 16, 16)] = pv
            sel_st[pl.ds(r * K + vi * 16, 16)] = idxv

    start(0, dbuf0, sem_a)

    def outer(gg, _):
        r0 = 2 * gg
        r1 = r0 + 1
        wait(r0, dbuf0, sem_a)
        start(r1, dbuf1, sem_b)
        process(r0, dbuf0)
        wait(r1, dbuf1, sem_b)

        @pl.when(r1 + 1 < RPW)
        def _():
            start(r1 + 1, dbuf0, sem_a)

        process(r1, dbuf1)
        return 0

    lax.fori_loop(0, RPW // 2, outer, 0)
    pltpu.sync_copy(psel_st, psel_hbm.at[pl.ds(row0 * K, RPW * K)])
    pltpu.sync_copy(sel_st, sel_hbm.at[pl.ds(row0 * K, RPW * K)])


def _topk_sc(dist, pmax, tau0):
    mesh = plsc.VectorSubcoreMesh(core_axis_name="c", subcore_axis_name="s")
    f = functools.partial(
        pl.kernel, mesh=mesh,
        compiler_params=pltpu.CompilerParams(needs_layout_passes=False),
        out_type=(jax.ShapeDtypeStruct((N * K,), jnp.float32),
                  jax.ShapeDtypeStruct((N * K,), jnp.int32)),
        scratch_types=[
            pltpu.VMEM((NE,), jnp.float32),
            pltpu.VMEM((NE,), jnp.float32),
            pltpu.VMEM((RPW,), jnp.float32),
            pltpu.VMEM((RPW,), jnp.float32),
            pltpu.VMEM((CAP_C,), jnp.int32),
            pltpu.VMEM((CAP_C,), jnp.int32),
            pltpu.VMEM((RPW * K,), jnp.float32),
            pltpu.VMEM((RPW * K,), jnp.int32),
            pltpu.SemaphoreType.DMA,
            pltpu.SemaphoreType.DMA,
        ],
    )(_topk_body)
    psel, sel = f(dist, pmax, tau0)
    return psel.reshape(N, K), sel.reshape(N, K)


def _sample_body(psel_ref, sel_ref, g_ref, out_ref):
    psel = psel_ref[...]
    s = jnp.sum(psel, axis=1, keepdims=True)
    pn = psel / s
    logits = jnp.log(jnp.clip(pn, 1e-20, None))
    w = logits + g_ref[...]
    mx = jnp.max(w, axis=1, keepdims=True)
    iota = lax.broadcasted_iota(jnp.int32, w.shape, 1)
    samp = jnp.min(jnp.where(w == mx, iota, K), axis=1, keepdims=True)
    ind = jnp.sum(jnp.where(iota == samp, sel_ref[...], 0), axis=1,
                  keepdims=True)
    out_ref[...] = ind


def _sample(psel, sel, gum):
    return pl.pallas_call(
        _sample_body,
        out_shape=jax.ShapeDtypeStruct((N, 1), jnp.int32),
    )(psel, sel, gum)


def _gather_body(ind_hbm, embt_hbm, x_hbm, q_hbm, part_hbm,
                 idxv, rows, xv, qv, accv, sem):
    wid = lax.axis_index("s") * 2 + lax.axis_index("c")

    def chunk(c, acc):
        base = wid * RPW + c * CHUNK
        pltpu.sync_copy(ind_hbm.at[pl.ds(base, CHUNK)], idxv)
        pltpu.async_copy(embt_hbm.at[idxv], rows, sem).wait()
        pltpu.sync_copy(x_hbm.at[pl.ds(base * DIM, CHUNK * DIM)], xv)

        def vstep(k, acc):
            r = k // 16
            seg = k % 16
            qrow = rows[r, pl.ds(seg * 16, 16)]
            x = xv[pl.ds(k * 16, 16)]
            dlt = qrow - x
            qv[pl.ds(k * 16, 16)] = x + dlt
            return acc + dlt * dlt

        acc = lax.fori_loop(0, CHUNK * 16, vstep, acc)
        pltpu.sync_copy(qv, q_hbm.at[pl.ds(base * DIM, CHUNK * DIM)])
        return acc

    acc = lax.fori_loop(0, RPW // CHUNK, chunk, jnp.zeros((16,), jnp.float32))
    accv[...] = acc
    pltpu.sync_copy(accv, part_hbm.at[wid])


def _gather_sc(ind, embt, xflat):
    mesh = plsc.VectorSubcoreMesh(core_axis_name="c", subcore_axis_name="s")
    f = functools.partial(
        pl.kernel, mesh=mesh,
        compiler_params=pltpu.CompilerParams(needs_layout_passes=False),
        out_type=(jax.ShapeDtypeStruct((N * DIM,), jnp.float32),
                  jax.ShapeDtypeStruct((NW, 16), jnp.float32)),
        scratch_types=[
            pltpu.VMEM((CHUNK,), jnp.int32),
            pltpu.VMEM((CHUNK, DIM), jnp.float32),
            pltpu.VMEM((CHUNK * DIM,), jnp.float32),
            pltpu.VMEM((CHUNK * DIM,), jnp.float32),
            pltpu.VMEM((16,), jnp.float32),
            pltpu.SemaphoreType.DMA,
        ],
    )(_gather_body)
    return f(ind, embt, xflat)


def kernel(input_lr, embed):
    flatten = input_lr.reshape(-1, DIM)
    dist, pmax, tau0 = _dist_pmax(flatten, embed)
    psel, sel = _topk_sc(dist, pmax.reshape(-1), tau0.reshape(-1))
    gum = jax.random.gumbel(jax.random.key(42), (N, K), jnp.float32)
    ind = _sample(psel, sel, gum)
    embt = embed.T
    qflat, partials = _gather_sc(ind.reshape(-1), embt, flatten.reshape(-1))
    quantize = qflat.reshape(input_lr.shape)
    diff = jnp.sum(partials) / jnp.float32(N * DIM)
    embed_ind = ind.reshape(input_lr.shape[:-1])
    return (quantize, diff, embed_ind, dist)
